# Initial kernel scaffold; baseline (speedup 1.0000x reference)
import jax
import jax.numpy as jnp
from jax.experimental import pallas as pl


def kernel(pred, target):
    raise NotImplementedError("write your pallas kernel here")


# speedup vs baseline: 10.9297x; 10.9297x over previous
import functools

import jax
import jax.numpy as jnp
import numpy as np
from jax.experimental import pallas as pl
from jax.experimental.pallas import tpu as pltpu

ALPHA, BETA, GAMMA, EPSILON = 1.0, 0.15, 0.05, 0.1
C1 = (0.01 * 255.0) ** 2
C2 = (0.03 * 255.0) ** 2

import ml_dtypes

_G = np.exp(-((np.arange(11) - 5.0) ** 2) / (2.0 * 1.5 ** 2))
_G = _G / _G.sum()
_W2D = np.outer(_G, _G).astype(np.float32)
_W2D_BF = _W2D.astype(ml_dtypes.bfloat16).astype(np.float32)
_WB = [[float(_W2D_BF[a, b]) for b in range(11)] for a in range(11)]

_R = 128
_HALO = 16
_NTERMS = 10


def _quant255(x):
    x = jnp.clip(x, -1.0, 1.0)
    x01 = jnp.clip((x + 1.0) * 0.5, 0.0, 1.0)
    return jnp.floor(jnp.clip(x01 * 255.0, 0.0, 255.0))


_SW = 256
_SRD = 384


def _conv2d_strip(x):
    xb = [x[:, b:b + _SW] for b in range(11)]
    acc = None
    for a in range(11):
        ha = None
        for b in range(11):
            t = _WB[a][b] * xb[b]
            ha = t if ha is None else ha + t
        hs = ha[a:a + _R]
        acc = hs if acc is None else acc + hs
    return acc


def _bf(x):
    return x.astype(jnp.bfloat16).astype(jnp.float32)


def _partials_kernel(pred_ref, pred_halo_ref, targ_ref, targ_halo_ref, out_ref,
                     p8_ref, t8_ref, *, h, w):
    j = pl.program_id(1)
    ow = w - 10
    oh = h - 10

    p_ext = jnp.concatenate([pred_ref[0], pred_halo_ref[0]], axis=0)
    t_ext = jnp.concatenate([targ_ref[0], targ_halo_ref[0]], axis=0)
    p = p_ext[:_R]
    t = t_ext[:_R]

    d_ext = p_ext[:_R + 1] - t_ext[:_R + 1]
    d = d_ext[:_R]
    s_absd = jnp.sum(jnp.abs(d))
    s_p = jnp.sum(p)
    s_t = jnp.sum(t)
    s_p2 = jnp.sum(p * p)
    s_t2 = jnp.sum(t * t)
    s_rp = jnp.sum(jnp.maximum(jnp.abs(p) - 1.0, 0.0))

    gxv = jnp.abs(d_ext[0:_R] - d_ext[1:_R + 1])
    gx_mask = jax.lax.broadcasted_iota(jnp.int32, (_R, w), 0) < (h - 1 - j * _R)
    s_gx = jnp.sum(jnp.where(gx_mask, gxv, 0.0))

    s_gy = jnp.sum(jnp.abs(d[:, :w - 1] - d[:, 1:]))

    p8 = _quant255(p_ext)
    t8 = _quant255(t_ext)
    dq = p8[:_R] - t8[:_R]
    s_mse8 = jnp.sum(dq * dq)

    p8_ref[:, 0:w] = p8
    t8_ref[:, 0:w] = t8
    p8_ref[:, w:w + _SW] = jnp.zeros((_R + _HALO, _SW), jnp.float32)
    t8_ref[:, w:w + _SW] = jnp.zeros((_R + _HALO, _SW), jnp.float32)
    nstrip = w // _SW
    row_ok = jax.lax.broadcasted_iota(jnp.int32, (_R, _SW), 0) < (oh - j * _R)

    def sbody(s, acc):
        off = s * _SW
        xp = p8_ref[0:_R + 10, pl.ds(off, _SRD)]
        xt = t8_ref[0:_R + 10, pl.ds(off, _SRD)]
        mu1 = _conv2d_strip(xp)
        mu2 = _conv2d_strip(xt)
        e11 = _conv2d_strip(_bf(xp * xp))
        e22 = _conv2d_strip(_bf(xt * xt))
        e12 = _conv2d_strip(_bf(xp * xt))
        mu1_sq = mu1 * mu1
        mu2_sq = mu2 * mu2
        mu1_mu2 = mu1 * mu2
        sigma1 = e11 - mu1_sq
        sigma2 = e22 - mu2_sq
        sigma12 = e12 - mu1_mu2
        ssim_map = ((2.0 * mu1_mu2 + C1) * (2.0 * sigma12 + C2)) / (
            (mu1_sq + mu2_sq + C1) * (sigma1 + sigma2 + C2))
        col_ok = (off + jax.lax.broadcasted_iota(jnp.int32, (_R, _SW), 1)) < ow
        return acc + jnp.sum(jnp.where(row_ok & col_ok, ssim_map, 0.0))

    s_ssim = jax.lax.fori_loop(0, nstrip, sbody, jnp.float32(0.0))

    vals = [s_absd, s_gx, s_gy, s_p, s_t, s_p2, s_t2, s_rp, s_mse8, s_ssim]
    lane = jax.lax.broadcasted_iota(jnp.int32, (1, 128), 1)
    vec = jnp.zeros((1, 128), jnp.float32)
    for idx, v in enumerate(vals):
        vec = jnp.where(lane == idx, v, vec)
    out_ref[...] = vec[None, None]


def _combine_kernel(part_ref, tot_ref, psnr_ref, ssim_ref, *, b, h, w):
    n = float(h * w)
    parts = part_ref[...]
    pp = jnp.sum(parts, axis=(1, 2))
    lane = jax.lax.broadcasted_iota(jnp.int32, (b, 128), 1)

    def pick(k):
        return jnp.sum(jnp.where(lane == k, pp, 0.0), axis=1, keepdims=True)

    def su(x):
        return jnp.sum(x, axis=0, keepdims=True)

    s_absd, s_gx, s_gy = pick(0), pick(1), pick(2)
    s_p, s_t, s_p2, s_t2 = pick(3), pick(4), pick(5), pick(6)
    s_rp, s_mse8, s_ssim = pick(7), pick(8), pick(9)

    l1 = su(s_absd) / (b * n)
    grad = su(s_gx) / (b * (h - 1) * w) + su(s_gy) / (b * h * (w - 1))

    pm = s_p / n
    tm = s_t / n
    energy = su((pm - tm) ** 2) / b
    ps = jnp.sqrt(jnp.maximum(s_p2 - n * pm * pm, 0.0) / (n - 1.0))
    ts = jnp.sqrt(jnp.maximum(s_t2 - n * tm * tm, 0.0) / (n - 1.0))
    dist = su((ps - ts) ** 2) / b
    range_pen = su(s_rp) / (b * n)
    phys = energy + 0.5 * dist + 0.1 * range_pen

    mse = s_mse8 / n
    inv_ln10 = 1.0 / float(np.log(10.0))
    psnr = 10.0 * inv_ln10 * jnp.log((255.0 ** 2) / jnp.maximum(mse, 1e-12))
    psnr = jnp.where(mse == 0.0, 100.0, psnr)
    psnr_mean = su(psnr) / b

    ssim_pi = s_ssim / float((h - 10) * (w - 10))
    ssim_mean = jnp.clip(su(ssim_pi) / b, 0.0, 1.0)

    total = (ALPHA * l1 + BETA * grad + GAMMA * phys +
             EPSILON * (1.0 - ssim_mean))
    tot_ref[...] = total
    psnr_ref[...] = psnr_mean
    ssim_ref[...] = ssim_mean


@jax.jit
def kernel(pred, target):
    b, c, h, w = pred.shape
    bc = b * c
    p = pred.reshape(bc, h, w)
    t = target.reshape(bc, h, w)
    nblk = h // _R
    nhalo = h // _HALO

    def cur_map(i, j):
        return (i, j, 0)

    def halo_map(i, j):
        return (i, jnp.minimum((j + 1) * (_R // _HALO), nhalo - 1), 0)

    partials = pl.pallas_call(
        functools.partial(_partials_kernel, h=h, w=w),
        grid=(bc, nblk),
        in_specs=[
            pl.BlockSpec((1, _R, w), cur_map),
            pl.BlockSpec((1, _HALO, w), halo_map),
            pl.BlockSpec((1, _R, w), cur_map),
            pl.BlockSpec((1, _HALO, w), halo_map),
        ],
        out_specs=pl.BlockSpec((1, 1, 1, 128), lambda i, j: (i, j, 0, 0)),
        out_shape=jax.ShapeDtypeStruct((bc, nblk, 1, 128), jnp.float32),
        scratch_shapes=[
            pltpu.VMEM((_R + _HALO, w + _SW), jnp.float32),
            pltpu.VMEM((_R + _HALO, w + _SW), jnp.float32),
        ],
        compiler_params=pltpu.CompilerParams(
            dimension_semantics=("parallel", "parallel"),
            vmem_limit_bytes=100 * 1024 * 1024,
        ),
    )(p, p, t, t)

    total, psnr_mean, ssim_mean = pl.pallas_call(
        functools.partial(_combine_kernel, b=bc, h=h, w=w),
        out_shape=[jax.ShapeDtypeStruct((1, 1), jnp.float32)] * 3,
    )(partials)

    return total[0, 0], psnr_mean[0, 0], ssim_mean[0, 0]


# speedup vs baseline: 83.5001x; 7.6397x over previous
import functools

import jax
import jax.numpy as jnp
import numpy as np
from jax.experimental import pallas as pl
from jax.experimental.pallas import tpu as pltpu

ALPHA, BETA, GAMMA, EPSILON = 1.0, 0.15, 0.05, 0.1
C1 = (0.01 * 255.0) ** 2
C2 = (0.03 * 255.0) ** 2

import ml_dtypes

_G = np.exp(-((np.arange(11) - 5.0) ** 2) / (2.0 * 1.5 ** 2))
_G = _G / _G.sum()
_W2D = np.outer(_G, _G).astype(np.float32)
_W2D_BF = _W2D.astype(ml_dtypes.bfloat16).astype(np.float32)
_WB = [[float(_W2D_BF[a, b]) for b in range(11)] for a in range(11)]

_R = 128
_HALO = 16
_NTERMS = 10


def _quant255(x):
    x = jnp.clip(x, -1.0, 1.0)
    x01 = jnp.clip((x + 1.0) * 0.5, 0.0, 1.0)
    return jnp.floor(jnp.clip(x01 * 255.0, 0.0, 255.0))


def _build_vmat(r):
    k = r + 16
    v = np.zeros((11 * r, k), np.float32)
    for b in range(11):
        for row in range(r):
            for a in range(11):
                v[b * r + row, row + a] = _W2D_BF[a, b]
    return v


def _partials_kernel(pred_ref, pred_halo_ref, targ_ref, targ_halo_ref, v_ref,
                     out_ref, z_ref, *, h, w):
    j = pl.program_id(1)
    ow = w - 10
    oh = h - 10

    p_ext = jnp.concatenate([pred_ref[0], pred_halo_ref[0]], axis=0)
    t_ext = jnp.concatenate([targ_ref[0], targ_halo_ref[0]], axis=0)
    p = p_ext[:_R]
    t = t_ext[:_R]

    d_ext = p_ext[:_R + 1] - t_ext[:_R + 1]
    d = d_ext[:_R]
    s_absd = jnp.sum(jnp.abs(d))
    s_p = jnp.sum(p)
    s_t = jnp.sum(t)
    s_p2 = jnp.sum(p * p)
    s_t2 = jnp.sum(t * t)
    s_rp = jnp.sum(jnp.maximum(jnp.abs(p) - 1.0, 0.0))

    gxv = jnp.abs(d_ext[0:_R] - d_ext[1:_R + 1])
    gx_mask = jax.lax.broadcasted_iota(jnp.int32, (_R, w), 0) < (h - 1 - j * _R)
    s_gx = jnp.sum(jnp.where(gx_mask, gxv, 0.0))

    s_gy = jnp.sum(jnp.abs(d[:, :w - 1] - d[:, 1:]))

    p8 = _quant255(p_ext)
    t8 = _quant255(t_ext)
    dq = p8[:_R] - t8[:_R]
    s_mse8 = jnp.sum(dq * dq)

    v = v_ref[...]
    z_ref[:, w:w + 128] = jnp.zeros((11 * _R, 128), jnp.float32)
    sigs = [p8.astype(jnp.bfloat16), t8.astype(jnp.bfloat16),
            (p8 * p8).astype(jnp.bfloat16), (t8 * t8).astype(jnp.bfloat16),
            (p8 * t8).astype(jnp.bfloat16)]
    outs = []
    for sig in sigs:
        z_ref[:, 0:w] = jnp.dot(v, sig, preferred_element_type=jnp.float32)
        acc = None
        for b in range(11):
            zb = z_ref[_R * b:_R * (b + 1), b:b + w]
            acc = zb if acc is None else acc + zb
        outs.append(acc)
    mu1, mu2, e11, e22, e12 = outs

    mu1_sq = mu1 * mu1
    mu2_sq = mu2 * mu2
    mu1_mu2 = mu1 * mu2
    sigma1 = e11 - mu1_sq
    sigma2 = e22 - mu2_sq
    sigma12 = e12 - mu1_mu2
    ssim_map = ((2.0 * mu1_mu2 + C1) * (2.0 * sigma12 + C2)) / (
        (mu1_sq + mu2_sq + C1) * (sigma1 + sigma2 + C2))
    row_ok = jax.lax.broadcasted_iota(jnp.int32, (_R, w), 0) < (oh - j * _R)
    col_ok = jax.lax.broadcasted_iota(jnp.int32, (_R, w), 1) < ow
    s_ssim = jnp.sum(jnp.where(row_ok & col_ok, ssim_map, 0.0))

    vals = [s_absd, s_gx, s_gy, s_p, s_t, s_p2, s_t2, s_rp, s_mse8, s_ssim]
    lane = jax.lax.broadcasted_iota(jnp.int32, (1, 128), 1)
    vec = jnp.zeros((1, 128), jnp.float32)
    for idx, v in enumerate(vals):
        vec = jnp.where(lane == idx, v, vec)
    out_ref[...] = vec[None, None]


def _combine_kernel(part_ref, tot_ref, psnr_ref, ssim_ref, *, b, h, w):
    n = float(h * w)
    parts = part_ref[...]
    pp = jnp.sum(parts, axis=(1, 2))
    lane = jax.lax.broadcasted_iota(jnp.int32, (b, 128), 1)

    def pick(k):
        return jnp.sum(jnp.where(lane == k, pp, 0.0), axis=1, keepdims=True)

    def su(x):
        return jnp.sum(x, axis=0, keepdims=True)

    s_absd, s_gx, s_gy = pick(0), pick(1), pick(2)
    s_p, s_t, s_p2, s_t2 = pick(3), pick(4), pick(5), pick(6)
    s_rp, s_mse8, s_ssim = pick(7), pick(8), pick(9)

    l1 = su(s_absd) / (b * n)
    grad = su(s_gx) / (b * (h - 1) * w) + su(s_gy) / (b * h * (w - 1))

    pm = s_p / n
    tm = s_t / n
    energy = su((pm - tm) ** 2) / b
    ps = jnp.sqrt(jnp.maximum(s_p2 - n * pm * pm, 0.0) / (n - 1.0))
    ts = jnp.sqrt(jnp.maximum(s_t2 - n * tm * tm, 0.0) / (n - 1.0))
    dist = su((ps - ts) ** 2) / b
    range_pen = su(s_rp) / (b * n)
    phys = energy + 0.5 * dist + 0.1 * range_pen

    mse = s_mse8 / n
    inv_ln10 = 1.0 / float(np.log(10.0))
    psnr = 10.0 * inv_ln10 * jnp.log((255.0 ** 2) / jnp.maximum(mse, 1e-12))
    psnr = jnp.where(mse == 0.0, 100.0, psnr)
    psnr_mean = su(psnr) / b

    ssim_pi = s_ssim / float((h - 10) * (w - 10))
    ssim_mean = jnp.clip(su(ssim_pi) / b, 0.0, 1.0)

    total = (ALPHA * l1 + BETA * grad + GAMMA * phys +
             EPSILON * (1.0 - ssim_mean))
    tot_ref[...] = total
    psnr_ref[...] = psnr_mean
    ssim_ref[...] = ssim_mean


@jax.jit
def kernel(pred, target):
    b, c, h, w = pred.shape
    bc = b * c
    p = pred.reshape(bc, h, w)
    t = target.reshape(bc, h, w)
    nblk = h // _R
    nhalo = h // _HALO
    vmat = jnp.asarray(_build_vmat(_R), dtype=jnp.bfloat16)

    def cur_map(i, j):
        return (i, j, 0)

    def halo_map(i, j):
        return (i, jnp.minimum((j + 1) * (_R // _HALO), nhalo - 1), 0)

    partials = pl.pallas_call(
        functools.partial(_partials_kernel, h=h, w=w),
        grid=(bc, nblk),
        in_specs=[
            pl.BlockSpec((1, _R, w), cur_map),
            pl.BlockSpec((1, _HALO, w), halo_map),
            pl.BlockSpec((1, _R, w), cur_map),
            pl.BlockSpec((1, _HALO, w), halo_map),
            pl.BlockSpec((11 * _R, _R + _HALO), lambda i, j: (0, 0)),
        ],
        out_specs=pl.BlockSpec((1, 1, 1, 128), lambda i, j: (i, j, 0, 0)),
        out_shape=jax.ShapeDtypeStruct((bc, nblk, 1, 128), jnp.float32),
        scratch_shapes=[
            pltpu.VMEM((11 * _R, w + 128), jnp.float32),
        ],
        compiler_params=pltpu.CompilerParams(
            dimension_semantics=("parallel", "parallel"),
            vmem_limit_bytes=100 * 1024 * 1024,
        ),
    )(p, p, t, t, vmat)

    total, psnr_mean, ssim_mean = pl.pallas_call(
        functools.partial(_combine_kernel, b=bc, h=h, w=w),
        out_shape=[jax.ShapeDtypeStruct((1, 1), jnp.float32)] * 3,
    )(partials)

    return total[0, 0], psnr_mean[0, 0], ssim_mean[0, 0]
